# Initial kernel scaffold; baseline (speedup 1.0000x reference)
import jax
import jax.numpy as jnp
from jax.experimental import pallas as pl


def kernel(A_indices, A_values, H, weight, att_weight):
    raise NotImplementedError("write your pallas kernel here")


# speedup vs baseline: 3.2417x; 3.2417x over previous
import functools

import jax
import jax.numpy as jnp
from jax import lax
from jax.experimental import pallas as pl
from jax.experimental.pallas import tpu as pltpu
from jax.experimental.pallas import tpu_sc as plsc

N = 10000
E = 320000
D = 128
NUM_A = 3
NUM_C = 2

NC = 2
NS = 16
LANES = 16
NW = NC * NS

CHUNK = 128
EA = NUM_A * E
CPT = -(-EA // (NW * CHUNK))
TOT = NW * CPT * CHUNK
RPT = 624
REM = N - NS * RPT


def _mm_body(h_ref, m_ref, o_ref):
    o_ref[0] = jnp.dot(h_ref[...], m_ref[0], preferred_element_type=jnp.float32)


def _tc_matmul(H, M, bn=2000):
    return pl.pallas_call(
        _mm_body,
        grid=(NUM_A, N // bn),
        in_specs=[
            pl.BlockSpec((bn, D), lambda a, j: (j, 0)),
            pl.BlockSpec((1, D, D), lambda a, j: (a, 0, 0)),
        ],
        out_specs=pl.BlockSpec((1, bn, D), lambda a, j: (a, j, 0)),
        out_shape=jax.ShapeDtypeStruct((NUM_A, N, D), jnp.float32),
    )(H, M)


def _add_body(a_ref, b_ref, o_ref):
    o_ref[...] = a_ref[...] + b_ref[...]


def _tc_add(a, b, bn=2000):
    return pl.pallas_call(
        _add_body,
        grid=(N // bn,),
        in_specs=[
            pl.BlockSpec((bn, D), lambda j: (j, 0)),
            pl.BlockSpec((bn, D), lambda j: (j, 0)),
        ],
        out_specs=pl.BlockSpec((bn, D), lambda j: (j, 0)),
        out_shape=jax.ShapeDtypeStruct((N, D), jnp.float32),
    )(a, b)


def _sc_body(g_hbm, cols_hbm, rows_hbm, vals_hbm, zeros_hbm, out_hbm,
             idx_g, idx_s, vals_r, buf, shared, sem):
    c = lax.axis_index("c")
    s = lax.axis_index("s")
    t = c * NS + s

    pltpu.sync_copy(zeros_hbm.at[pl.ds(s * RPT, RPT)],
                    shared.at[pl.ds(s * RPT, RPT)])

    @pl.when(s == 0)
    def _():
        pltpu.sync_copy(zeros_hbm.at[pl.ds(NS * RPT, REM)],
                        shared.at[pl.ds(NS * RPT, REM)])

    plsc.subcore_barrier()

    base = t * CPT * CHUNK

    def chunk(q, carry):
        off = base + q * CHUNK
        pltpu.sync_copy(cols_hbm.at[pl.ds(off, CHUNK)], idx_g)
        pltpu.sync_copy(rows_hbm.at[pl.ds(off, CHUNK)], idx_s)
        pltpu.sync_copy(vals_hbm.at[pl.ds(off * LANES, CHUNK * LANES)], vals_r)
        pltpu.async_copy(g_hbm.at[idx_g], buf, sem).wait()

        def scale(e, carry2):
            v = vals_r[pl.ds(e * LANES, LANES)]
            for l in range(D // LANES):
                buf[e, pl.ds(l * LANES, LANES)] = (
                    buf[e, pl.ds(l * LANES, LANES)] * v)
            return carry2

        lax.fori_loop(0, CHUNK, scale, 0)
        pltpu.sync_copy(buf, shared.at[idx_s], add=True)
        return carry

    lax.fori_loop(0, CPT, chunk, 0)
    plsc.subcore_barrier()
    pltpu.sync_copy(shared.at[pl.ds(s * RPT, RPT)],
                    out_hbm.at[c, pl.ds(s * RPT, RPT)])

    @pl.when(s == 0)
    def _():
        pltpu.sync_copy(shared.at[pl.ds(NS * RPT, REM)],
                        out_hbm.at[c, pl.ds(NS * RPT, REM)])


_sc_spmm = pl.kernel(
    _sc_body,
    out_type=jax.ShapeDtypeStruct((NC, N, D), jnp.float32),
    mesh=plsc.VectorSubcoreMesh(core_axis_name="c", subcore_axis_name="s",
                                num_cores=NC, num_subcores=NS),
    scratch_types=[
        pltpu.VMEM((CHUNK,), jnp.int32),
        pltpu.VMEM((CHUNK,), jnp.int32),
        pltpu.VMEM((CHUNK * LANES,), jnp.float32),
        pltpu.VMEM((CHUNK, D), jnp.float32),
        pltpu.MemorySpace.VMEM_SHARED((N, D), jnp.float32),
        pltpu.SemaphoreType.DMA,
    ],
)


def kernel(A_indices, A_values, H, weight, att_weight):
    att = att_weight.mean(axis=1)
    score = jax.nn.softmax(att, axis=1)
    M = jnp.einsum("ca,cij->aij", score, weight) / NUM_C

    G = _tc_matmul(H, M)
    G = G.reshape(NUM_A * N, D)

    cols = (A_indices[:, 1, :]
            + (jnp.arange(NUM_A, dtype=jnp.int32) * N)[:, None]).reshape(-1)
    rows = A_indices[:, 0, :].reshape(-1)
    vals = A_values.reshape(-1)

    pad = TOT - EA
    ppos = jnp.arange(pad, dtype=jnp.int32)
    cols = jnp.concatenate([cols, (ppos * 97) % (NUM_A * N)])
    rows = jnp.concatenate([rows, (ppos * 89) % N])
    vals = jnp.concatenate([vals, jnp.zeros(pad, jnp.float32)])

    vals16 = jnp.repeat(vals, LANES)

    zeros = jnp.zeros((N, D), jnp.float32)
    partials = _sc_spmm(G, cols, rows, vals16, zeros)
    return _tc_add(partials[0], partials[1])


# speedup vs baseline: 5.1847x; 1.5994x over previous
import functools

import jax
import jax.numpy as jnp
from jax import lax
from jax.experimental import pallas as pl
from jax.experimental.pallas import tpu as pltpu
from jax.experimental.pallas import tpu_sc as plsc

N = 10000
E = 320000
D = 128
NUM_A = 3
NUM_C = 2

NC = 2
NS = 16
LANES = 16
NW = NC * NS

CHUNK = 128
EA = NUM_A * E
NBUF = 3
UNROLL = 6
CPT = 240
TOT = NW * CPT * CHUNK
RPT = 624
REM = N - NS * RPT


def _mm_body(h_ref, m_ref, o_ref):
    o_ref[0] = jnp.dot(h_ref[...], m_ref[0], preferred_element_type=jnp.float32)


def _tc_matmul(H, M, bn=2000):
    return pl.pallas_call(
        _mm_body,
        grid=(NUM_A, N // bn),
        in_specs=[
            pl.BlockSpec((bn, D), lambda a, j: (j, 0)),
            pl.BlockSpec((1, D, D), lambda a, j: (a, 0, 0)),
        ],
        out_specs=pl.BlockSpec((1, bn, D), lambda a, j: (a, j, 0)),
        out_shape=jax.ShapeDtypeStruct((NUM_A, N, D), jnp.float32),
    )(H, M)


def _add_body(a_ref, b_ref, o_ref):
    o_ref[...] = a_ref[...] + b_ref[...]


def _tc_add(a, b, bn=2000):
    return pl.pallas_call(
        _add_body,
        grid=(N // bn,),
        in_specs=[
            pl.BlockSpec((bn, D), lambda j: (j, 0)),
            pl.BlockSpec((bn, D), lambda j: (j, 0)),
        ],
        out_specs=pl.BlockSpec((bn, D), lambda j: (j, 0)),
        out_shape=jax.ShapeDtypeStruct((N, D), jnp.float32),
    )(a, b)


def _sc_body(g_hbm, cols_hbm, rows_hbm, vals_hbm, zeros_hbm, out_hbm,
             ig0, ig1, ig2, is0, is1, is2, vr0, vr1, vr2, bf0, bf1,
             shared, sem_i, sem_g, sem_s):
    idx_g = [ig0, ig1, ig2]
    idx_s = [is0, is1, is2]
    vals_r = [vr0, vr1, vr2]
    buf = [bf0, bf1]
    c = lax.axis_index("c")
    s = lax.axis_index("s")
    t = c * NS + s

    pltpu.sync_copy(zeros_hbm.at[pl.ds(s * RPT, RPT)],
                    shared.at[pl.ds(s * RPT, RPT)])

    @pl.when(s == 0)
    def _():
        pltpu.sync_copy(zeros_hbm.at[pl.ds(NS * RPT, REM)],
                        shared.at[pl.ds(NS * RPT, REM)])

    plsc.subcore_barrier()

    base = t * CPT * CHUNK

    def issue_idx(q, b):
        off = base + q * CHUNK
        pltpu.async_copy(cols_hbm.at[pl.ds(off, CHUNK)], idx_g[b],
                         sem_i.at[b])
        pltpu.async_copy(rows_hbm.at[pl.ds(off, CHUNK)], idx_s[b],
                         sem_i.at[b])
        pltpu.async_copy(vals_hbm.at[pl.ds(off * LANES, CHUNK * LANES)],
                         vals_r[b], sem_i.at[b])

    def wait_idx(b):
        pltpu.make_async_copy(cols_hbm.at[pl.ds(0, CHUNK)], idx_g[b],
                              sem_i.at[b]).wait()
        pltpu.make_async_copy(rows_hbm.at[pl.ds(0, CHUNK)], idx_s[b],
                              sem_i.at[b]).wait()
        pltpu.make_async_copy(vals_hbm.at[pl.ds(0, CHUNK * LANES)],
                              vals_r[b], sem_i.at[b]).wait()

    def issue_gather(bi, bb):
        pltpu.async_copy(g_hbm.at[idx_g[bi]], buf[bb], sem_g.at[bb])

    def wait_gather(bi, bb):
        pltpu.make_async_copy(g_hbm.at[idx_g[bi]], buf[bb],
                              sem_g.at[bb]).wait()

    def issue_scatter(bi, bb):
        pltpu.async_copy(buf[bb], shared.at[idx_s[bi]], sem_s.at[bb],
                         add=True)

    def wait_scatter(bi, bb):
        pltpu.make_async_copy(buf[bb], shared.at[idx_s[bi]],
                              sem_s.at[bb]).wait()

    def scale(bi, bb):
        vr = vals_r[bi]
        bf = buf[bb]

        @plsc.parallel_loop(0, CHUNK, unroll=2)
        def _(e):
            v = vr[pl.ds(e * LANES, LANES)]
            for l in range(D // LANES):
                bf[e, pl.ds(l * LANES, LANES)] = (
                    bf[e, pl.ds(l * LANES, LANES)] * v)

    issue_idx(0, 0)
    issue_idx(1, 1)
    wait_idx(0)
    issue_gather(0, 0)

    def group(g, carry):
        q0 = g * UNROLL
        for j in range(UNROLL):
            q = q0 + j
            bi = j % NBUF
            bb = j % 2

            @pl.when(q >= 1)
            def _():
                wait_scatter((j - 1) % NBUF, (j - 1) % 2)

            @pl.when(q + 2 < CPT)
            def _():
                issue_idx(q + 2, (j + 2) % NBUF)

            @pl.when(q + 1 < CPT)
            def _():
                wait_idx((j + 1) % NBUF)
                issue_gather((j + 1) % NBUF, (j + 1) % 2)

            wait_gather(bi, bb)
            scale(bi, bb)
            issue_scatter(bi, bb)
        return carry

    lax.fori_loop(0, CPT // UNROLL, group, 0)
    wait_scatter((CPT - 1) % NBUF, (CPT - 1) % 2)
    plsc.subcore_barrier()
    pltpu.sync_copy(shared.at[pl.ds(s * RPT, RPT)],
                    out_hbm.at[c, pl.ds(s * RPT, RPT)])

    @pl.when(s == 0)
    def _():
        pltpu.sync_copy(shared.at[pl.ds(NS * RPT, REM)],
                        out_hbm.at[c, pl.ds(NS * RPT, REM)])


_sc_spmm = pl.kernel(
    _sc_body,
    out_type=jax.ShapeDtypeStruct((NC, N, D), jnp.float32),
    mesh=plsc.VectorSubcoreMesh(core_axis_name="c", subcore_axis_name="s",
                                num_cores=NC, num_subcores=NS),
    scratch_types=(
        [pltpu.VMEM((CHUNK,), jnp.int32) for _ in range(NBUF)]
        + [pltpu.VMEM((CHUNK,), jnp.int32) for _ in range(NBUF)]
        + [pltpu.VMEM((CHUNK * LANES,), jnp.float32)
           for _ in range(NBUF)]
        + [pltpu.VMEM((CHUNK, D), jnp.float32) for _ in range(2)]
        + [pltpu.MemorySpace.VMEM_SHARED((N, D), jnp.float32)]
        + [pltpu.SemaphoreType.DMA((NBUF,)), pltpu.SemaphoreType.DMA((2,)),
           pltpu.SemaphoreType.DMA((2,))]
    ),
)


def kernel(A_indices, A_values, H, weight, att_weight):
    att = att_weight.mean(axis=1)
    score = jax.nn.softmax(att, axis=1)
    M = jnp.einsum("ca,cij->aij", score, weight) / NUM_C

    G = _tc_matmul(H, M)
    G = G.reshape(NUM_A * N, D)

    cols = (A_indices[:, 1, :]
            + (jnp.arange(NUM_A, dtype=jnp.int32) * N)[:, None]).reshape(-1)
    rows = A_indices[:, 0, :].reshape(-1)
    vals = A_values.reshape(-1)

    pad = TOT - EA
    ppos = jnp.arange(pad, dtype=jnp.int32)
    cols = jnp.concatenate([cols, (ppos * 97) % (NUM_A * N)])
    rows = jnp.concatenate([rows, (ppos * 89) % N])
    vals = jnp.concatenate([vals, jnp.zeros(pad, jnp.float32)])

    vals16 = jnp.repeat(vals, LANES)

    zeros = jnp.zeros((N, D), jnp.float32)
    partials = _sc_spmm(G, cols, rows, vals16, zeros)
    return _tc_add(partials[0], partials[1])


# speedup vs baseline: 11.6100x; 2.2393x over previous
import functools

import jax
import jax.numpy as jnp
from jax import lax
from jax.experimental import pallas as pl
from jax.experimental.pallas import tpu as pltpu
from jax.experimental.pallas import tpu_sc as plsc

N = 10000
E = 320000
D = 128
NUM_A = 3
NUM_C = 2

NC = 2
NS = 16
LANES = 16
NW = NC * NS

CHUNK = 128
EA = NUM_A * E
NBUF = 3
UNROLL = 6
CPT = 240
TOT = NW * CPT * CHUNK
RPT = 624
REM = N - NS * RPT


def _mm_body(h_ref, m_ref, o_ref):
    o_ref[0] = jnp.dot(h_ref[...], m_ref[0], preferred_element_type=jnp.float32)


def _tc_matmul(H, M, bn=2000):
    return pl.pallas_call(
        _mm_body,
        grid=(NUM_A, N // bn),
        in_specs=[
            pl.BlockSpec((bn, D), lambda a, j: (j, 0)),
            pl.BlockSpec((1, D, D), lambda a, j: (a, 0, 0)),
        ],
        out_specs=pl.BlockSpec((1, bn, D), lambda a, j: (a, j, 0)),
        out_shape=jax.ShapeDtypeStruct((NUM_A, N, D), jnp.float32),
    )(H, M)


def _add_body(a_ref, b_ref, o_ref):
    o_ref[...] = a_ref[...] + b_ref[...]


def _tc_add(a, b, bn=2000):
    return pl.pallas_call(
        _add_body,
        grid=(N // bn,),
        in_specs=[
            pl.BlockSpec((bn, D), lambda j: (j, 0)),
            pl.BlockSpec((bn, D), lambda j: (j, 0)),
        ],
        out_specs=pl.BlockSpec((bn, D), lambda j: (j, 0)),
        out_shape=jax.ShapeDtypeStruct((N, D), jnp.float32),
    )(a, b)


def _sc_body(g_hbm, cols_hbm, rows_hbm, vals_hbm, zeros_hbm, out_hbm,
             ig0, ig1, ig2, is0, is1, is2, vr0, vr1, vr2, bf0, bf1,
             shared, sem_i, sem_g, sem_s):
    idx_g = [ig0, ig1, ig2]
    idx_s = [is0, is1, is2]
    vals_r = [vr0, vr1, vr2]
    buf = [bf0, bf1]
    c = lax.axis_index("c")
    s = lax.axis_index("s")
    t = c * NS + s

    pltpu.sync_copy(zeros_hbm.at[pl.ds(s * RPT, RPT)],
                    shared.at[pl.ds(s * RPT, RPT)])

    @pl.when(s == 0)
    def _():
        pltpu.sync_copy(zeros_hbm.at[pl.ds(NS * RPT, REM)],
                        shared.at[pl.ds(NS * RPT, REM)])

    plsc.subcore_barrier()

    base = t * CPT * CHUNK

    def issue_idx(q, b):
        off = base + q * CHUNK
        pltpu.async_copy(cols_hbm.at[pl.ds(off, CHUNK)], idx_g[b],
                         sem_i.at[b])
        pltpu.async_copy(rows_hbm.at[pl.ds(off, CHUNK)], idx_s[b],
                         sem_i.at[b])
        pltpu.async_copy(vals_hbm.at[pl.ds(off, CHUNK)],
                         vals_r[b], sem_i.at[b])

    def wait_idx(b):
        pltpu.make_async_copy(cols_hbm.at[pl.ds(0, CHUNK)], idx_g[b],
                              sem_i.at[b]).wait()
        pltpu.make_async_copy(rows_hbm.at[pl.ds(0, CHUNK)], idx_s[b],
                              sem_i.at[b]).wait()
        pltpu.make_async_copy(vals_hbm.at[pl.ds(0, CHUNK)],
                              vals_r[b], sem_i.at[b]).wait()

    def issue_gather(bi, bb):
        pltpu.async_copy(g_hbm.at[idx_g[bi]], buf[bb], sem_g.at[bb])

    def wait_gather(bi, bb):
        pltpu.make_async_copy(g_hbm.at[idx_g[bi]], buf[bb],
                              sem_g.at[bb]).wait()

    def issue_scatter(bi, bb):
        pltpu.async_copy(buf[bb], shared.at[idx_s[bi]], sem_s.at[bb],
                         add=True)

    def wait_scatter(bi, bb):
        pltpu.make_async_copy(buf[bb], shared.at[idx_s[bi]],
                              sem_s.at[bb]).wait()

    def scale(bi, bb):
        vr = vals_r[bi]
        bf = buf[bb]

        @plsc.parallel_loop(0, CHUNK // LANES, unroll=1)
        def _(g):
            v16 = vr[pl.ds(g * LANES, LANES)]
            for j in range(LANES):
                e = g * LANES + j
                v = v16.at[jnp.full((LANES,), j, jnp.int32)].get(
                    mode="promise_in_bounds")
                for l in range(D // LANES):
                    bf[e, pl.ds(l * LANES, LANES)] = (
                        bf[e, pl.ds(l * LANES, LANES)] * v)

    issue_idx(0, 0)
    issue_idx(1, 1)
    wait_idx(0)
    issue_gather(0, 0)

    def group(g, carry):
        q0 = g * UNROLL
        for j in range(UNROLL):
            q = q0 + j
            bi = j % NBUF
            bb = j % 2

            @pl.when(q >= 1)
            def _():
                wait_scatter((j - 1) % NBUF, (j - 1) % 2)

            @pl.when(q + 2 < CPT)
            def _():
                issue_idx(q + 2, (j + 2) % NBUF)

            @pl.when(q + 1 < CPT)
            def _():
                wait_idx((j + 1) % NBUF)
                issue_gather((j + 1) % NBUF, (j + 1) % 2)

            wait_gather(bi, bb)
            scale(bi, bb)
            issue_scatter(bi, bb)
        return carry

    lax.fori_loop(0, CPT // UNROLL, group, 0)
    wait_scatter((CPT - 1) % NBUF, (CPT - 1) % 2)
    plsc.subcore_barrier()
    pltpu.sync_copy(shared.at[pl.ds(s * RPT, RPT)],
                    out_hbm.at[c, pl.ds(s * RPT, RPT)])

    @pl.when(s == 0)
    def _():
        pltpu.sync_copy(shared.at[pl.ds(NS * RPT, REM)],
                        out_hbm.at[c, pl.ds(NS * RPT, REM)])


_sc_spmm = pl.kernel(
    _sc_body,
    out_type=jax.ShapeDtypeStruct((NC, N, D), jnp.float32),
    mesh=plsc.VectorSubcoreMesh(core_axis_name="c", subcore_axis_name="s",
                                num_cores=NC, num_subcores=NS),
    scratch_types=(
        [pltpu.VMEM((CHUNK,), jnp.int32) for _ in range(NBUF)]
        + [pltpu.VMEM((CHUNK,), jnp.int32) for _ in range(NBUF)]
        + [pltpu.VMEM((CHUNK,), jnp.float32) for _ in range(NBUF)]
        + [pltpu.VMEM((CHUNK, D), jnp.float32) for _ in range(2)]
        + [pltpu.MemorySpace.VMEM_SHARED((N, D), jnp.float32)]
        + [pltpu.SemaphoreType.DMA((NBUF,)), pltpu.SemaphoreType.DMA((2,)),
           pltpu.SemaphoreType.DMA((2,))]
    ),
)


def kernel(A_indices, A_values, H, weight, att_weight):
    att = att_weight.mean(axis=1)
    score = jax.nn.softmax(att, axis=1)
    M = jnp.einsum("ca,cij->aij", score, weight) / NUM_C

    G = _tc_matmul(H, M)
    G = G.reshape(NUM_A * N, D)

    cols = (A_indices[:, 1, :]
            + (jnp.arange(NUM_A, dtype=jnp.int32) * N)[:, None]).reshape(-1)
    rows = A_indices[:, 0, :].reshape(-1)
    vals = A_values.reshape(-1)

    pad = TOT - EA
    ppos = jnp.arange(pad, dtype=jnp.int32)
    cols = jnp.concatenate([cols, (ppos * 97) % (NUM_A * N)])
    rows = jnp.concatenate([rows, (ppos * 89) % N])
    vals = jnp.concatenate([vals, jnp.zeros(pad, jnp.float32)])

    zeros = jnp.zeros((N, D), jnp.float32)
    partials = _sc_spmm(G, cols, rows, vals, zeros)
    return _tc_add(partials[0], partials[1])


# speedup vs baseline: 15.1946x; 1.3087x over previous
import jax
import jax.numpy as jnp
from jax import lax
from jax.experimental import pallas as pl
from jax.experimental.pallas import tpu as pltpu
from jax.experimental.pallas import tpu_sc as plsc

N = 10000
E = 320000
D = 128
NUM_A = 3
NUM_C = 2

NC = 2
NS = 16
LANES = 16
NW = NC * NS

CHUNK = 128
EA = NUM_A * E
NCH = EA // CHUNK
CPA = E // CHUNK
NBUF = 3
CPT = NCH // NW
NREM = NCH - CPT * NW
RPT = 624
REM = N - NS * RPT


def _mm_body(h_ref, m_ref, o_ref):
    o_ref[...] = jnp.dot(h_ref[...], m_ref[0],
                         preferred_element_type=jnp.float32)


def _tc_matmul(H, M, bn=2000):
    nb = N // bn
    return pl.pallas_call(
        _mm_body,
        grid=(NUM_A, nb),
        in_specs=[
            pl.BlockSpec((bn, D), lambda a, j: (j, 0)),
            pl.BlockSpec((1, D, D), lambda a, j: (a, 0, 0)),
        ],
        out_specs=pl.BlockSpec((bn, D), lambda a, j: (a * nb + j, 0)),
        out_shape=jax.ShapeDtypeStruct((NUM_A * N, D), jnp.float32),
    )(H, M)


def _add_body(a_ref, b_ref, o_ref):
    o_ref[...] = a_ref[0] + b_ref[0]


def _tc_add(partials, bn=2000):
    return pl.pallas_call(
        _add_body,
        grid=(N // bn,),
        in_specs=[
            pl.BlockSpec((1, bn, D), lambda j: (0, j, 0)),
            pl.BlockSpec((1, bn, D), lambda j: (1, j, 0)),
        ],
        out_specs=pl.BlockSpec((bn, D), lambda j: (j, 0)),
        out_shape=jax.ShapeDtypeStruct((N, D), jnp.float32),
    )(partials, partials)


def _sc_body(g_hbm, ai_hbm, av_hbm, zeros_hbm, out_hbm,
             ig0, ig1, ig2, is0, is1, is2, vr0, vr1, vr2, bf0, bf1, bf2,
             shared, sem_i, sem_g, sem_s):
    idx_g = [ig0, ig1, ig2]
    idx_s = [is0, is1, is2]
    vals_r = [vr0, vr1, vr2]
    buf = [bf0, bf1, bf2]
    c = lax.axis_index("c")
    s = lax.axis_index("s")
    t = c * NS + s

    pltpu.sync_copy(zeros_hbm.at[pl.ds(s * RPT, RPT)],
                    shared.at[pl.ds(s * RPT, RPT)])

    @pl.when(s == 0)
    def _():
        pltpu.sync_copy(zeros_hbm.at[pl.ds(NS * RPT, REM)],
                        shared.at[pl.ds(NS * RPT, REM)])

    plsc.subcore_barrier()

    def chunk_id(q):
        return q * NW + t

    def adj_of(k):
        return ((k >= CPA).astype(jnp.int32)
                + (k >= 2 * CPA).astype(jnp.int32))

    def issue_idx(q, b):
        k = chunk_id(q)
        a = adj_of(k)
        w = k - a * CPA
        rows_off = a * (2 * E) + w * CHUNK
        pltpu.async_copy(ai_hbm.at[pl.ds(rows_off + E, CHUNK)], idx_g[b],
                         sem_i.at[b])
        pltpu.async_copy(ai_hbm.at[pl.ds(rows_off, CHUNK)], idx_s[b],
                         sem_i.at[b])
        pltpu.async_copy(av_hbm.at[pl.ds(k * CHUNK, CHUNK)],
                         vals_r[b], sem_i.at[b])

    def wait_idx(b):
        pltpu.make_async_copy(ai_hbm.at[pl.ds(0, CHUNK)], idx_g[b],
                              sem_i.at[b]).wait()
        pltpu.make_async_copy(ai_hbm.at[pl.ds(0, CHUNK)], idx_s[b],
                              sem_i.at[b]).wait()
        pltpu.make_async_copy(av_hbm.at[pl.ds(0, CHUNK)],
                              vals_r[b], sem_i.at[b]).wait()

    def add_base(q, b):
        aN = adj_of(chunk_id(q)) * N
        for l in range(CHUNK // LANES):
            idx_g[b][pl.ds(l * LANES, LANES)] = (
                idx_g[b][pl.ds(l * LANES, LANES)] + aN)

    def issue_gather(bi, bb):
        pltpu.async_copy(g_hbm.at[idx_g[bi]], buf[bb], sem_g.at[bb])

    def wait_gather(bi, bb):
        pltpu.make_async_copy(g_hbm.at[idx_g[bi]], buf[bb],
                              sem_g.at[bb]).wait()

    def issue_scatter(bi, bb):
        pltpu.async_copy(buf[bb], shared.at[idx_s[bi]], sem_s.at[bb],
                         add=True)

    def wait_scatter(bi, bb):
        pltpu.make_async_copy(buf[bb], shared.at[idx_s[bi]],
                              sem_s.at[bb]).wait()

    def scale(bi, bb):
        vr = vals_r[bi]
        bf = buf[bb]

        @plsc.parallel_loop(0, CHUNK // LANES, unroll=1)
        def _(g):
            v16 = vr[pl.ds(g * LANES, LANES)]
            for j in range(LANES):
                e = g * LANES + j
                v = v16.at[jnp.full((LANES,), j, jnp.int32)].get(
                    mode="promise_in_bounds")
                for l in range(D // LANES):
                    bf[e, pl.ds(l * LANES, LANES)] = (
                        bf[e, pl.ds(l * LANES, LANES)] * v)

    issue_idx(0, 0)
    issue_idx(1, 1)
    wait_idx(0)
    add_base(0, 0)
    issue_gather(0, 0)

    def group(g, carry):
        q0 = g * NBUF
        for j in range(NBUF):
            q = q0 + j
            b = j
            bm2 = (j - 2) % NBUF
            bp1 = (j + 1) % NBUF
            bp2 = (j + 2) % NBUF

            @pl.when(q >= 2)
            def _():
                wait_scatter(bm2, bm2)

            @pl.when(q + 2 < CPT)
            def _():
                issue_idx(q + 2, bp2)

            @pl.when(q + 1 < CPT)
            def _():
                wait_idx(bp1)
                add_base(q + 1, bp1)
                issue_gather(bp1, bp1)

            wait_gather(b, b)
            scale(b, b)
            issue_scatter(b, b)
        return carry

    lax.fori_loop(0, CPT // NBUF, group, 0)
    wait_scatter((CPT - 2) % NBUF, (CPT - 2) % NBUF)
    wait_scatter((CPT - 1) % NBUF, (CPT - 1) % NBUF)

    @pl.when(t < NREM)
    def _():
        issue_idx(CPT, 0)
        wait_idx(0)
        add_base(CPT, 0)
        issue_gather(0, 0)
        wait_gather(0, 0)
        scale(0, 0)
        issue_scatter(0, 0)
        wait_scatter(0, 0)

    plsc.subcore_barrier()
    pltpu.sync_copy(shared.at[pl.ds(s * RPT, RPT)],
                    out_hbm.at[c, pl.ds(s * RPT, RPT)])

    @pl.when(s == 0)
    def _():
        pltpu.sync_copy(shared.at[pl.ds(NS * RPT, REM)],
                        out_hbm.at[c, pl.ds(NS * RPT, REM)])


_sc_spmm = pl.kernel(
    _sc_body,
    out_type=jax.ShapeDtypeStruct((NC, N, D), jnp.float32),
    mesh=plsc.VectorSubcoreMesh(core_axis_name="c", subcore_axis_name="s",
                                num_cores=NC, num_subcores=NS),
    scratch_types=(
        [pltpu.VMEM((CHUNK,), jnp.int32) for _ in range(NBUF)]
        + [pltpu.VMEM((CHUNK,), jnp.int32) for _ in range(NBUF)]
        + [pltpu.VMEM((CHUNK,), jnp.float32) for _ in range(NBUF)]
        + [pltpu.VMEM((CHUNK, D), jnp.float32) for _ in range(NBUF)]
        + [pltpu.MemorySpace.VMEM_SHARED((N, D), jnp.float32)]
        + [pltpu.SemaphoreType.DMA((NBUF,)) for _ in range(3)]
    ),
)


def kernel(A_indices, A_values, H, weight, att_weight):
    att = att_weight.mean(axis=1)
    score = jax.nn.softmax(att, axis=1)
    M = jnp.einsum("ca,cij->aij", score, weight) / NUM_C

    G = _tc_matmul(H, M)

    zeros = jnp.zeros((N, D), jnp.float32)
    partials = _sc_spmm(G, A_indices.reshape(-1), A_values.reshape(-1),
                        zeros)
    return _tc_add(partials)


# speedup vs baseline: 15.3354x; 1.0093x over previous
import jax
import jax.numpy as jnp
from jax import lax
from jax.experimental import pallas as pl
from jax.experimental.pallas import tpu as pltpu
from jax.experimental.pallas import tpu_sc as plsc

N = 10000
E = 320000
D = 128
NUM_A = 3
NUM_C = 2

NC = 2
NS = 16
LANES = 16
NW = NC * NS

CHUNK = 128
EA = NUM_A * E
NCH = EA // CHUNK
CPA = E // CHUNK
NBUF = 3
CPT = NCH // NW
NREM = NCH - CPT * NW
RPT = 624
REM = N - NS * RPT


def _mm_body(h_ref, m_ref, o_ref):
    o_ref[...] = jnp.dot(h_ref[...], m_ref[0],
                         preferred_element_type=jnp.float32)


def _tc_matmul(H, M, bn=2000):
    nb = N // bn
    return pl.pallas_call(
        _mm_body,
        grid=(nb, NUM_A),
        in_specs=[
            pl.BlockSpec((bn, D), lambda j, a: (j, 0)),
            pl.BlockSpec((1, D, D), lambda j, a: (a, 0, 0)),
        ],
        out_specs=pl.BlockSpec((bn, D), lambda j, a: (a * nb + j, 0)),
        out_shape=jax.ShapeDtypeStruct((NUM_A * N, D), jnp.float32),
    )(H, M)


def _repack_body(ai_ref, rows_ref, cols_ref):
    a = pl.program_id(0)
    rows_ref[0, 0] = ai_ref[0, 0]
    cols_ref[0, 0] = ai_ref[0, 1] + a * N


def _tc_repack(A_indices, ne=10):
    eb = E // ne
    i32 = jnp.int32
    return pl.pallas_call(
        _repack_body,
        grid=(NUM_A, ne),
        in_specs=[pl.BlockSpec((1, 2, eb), lambda a, j: (a, 0, j))],
        out_specs=[pl.BlockSpec((1, 1, eb), lambda a, j: (a, 0, j))] * 2,
        out_shape=[jax.ShapeDtypeStruct((NUM_A, 1, E), i32),
                   jax.ShapeDtypeStruct((NUM_A, 1, E), i32)],
    )(A_indices)


def _add_body(a_ref, b_ref, o_ref):
    o_ref[...] = a_ref[0] + b_ref[0]


def _tc_add(partials, bn=2000):
    return pl.pallas_call(
        _add_body,
        grid=(N // bn,),
        in_specs=[
            pl.BlockSpec((1, bn, D), lambda j: (0, j, 0)),
            pl.BlockSpec((1, bn, D), lambda j: (1, j, 0)),
        ],
        out_specs=pl.BlockSpec((bn, D), lambda j: (j, 0)),
        out_shape=jax.ShapeDtypeStruct((N, D), jnp.float32),
    )(partials, partials)


def _sc_body(g_hbm, rows_hbm, cols_hbm, vals_hbm, zeros_hbm, out_hbm,
             ig0, ig1, ig2, is0, is1, is2, vr0, vr1, vr2, bf0, bf1, bf2,
             shared, sem_i, sem_g, sem_s):
    idx_g = [ig0, ig1, ig2]
    idx_s = [is0, is1, is2]
    vals_r = [vr0, vr1, vr2]
    buf = [bf0, bf1, bf2]
    c = lax.axis_index("c")
    s = lax.axis_index("s")
    t = c * NS + s

    pltpu.sync_copy(zeros_hbm.at[pl.ds(s * RPT, RPT)],
                    shared.at[pl.ds(s * RPT, RPT)])

    @pl.when(s == 0)
    def _():
        pltpu.sync_copy(zeros_hbm.at[pl.ds(NS * RPT, REM)],
                        shared.at[pl.ds(NS * RPT, REM)])

    plsc.subcore_barrier()

    def issue_idx(q, b):
        k = q * NW + t
        a = ((k >= CPA).astype(jnp.int32)
             + (k >= 2 * CPA).astype(jnp.int32))
        w = (k - a * CPA) * CHUNK
        pltpu.async_copy(cols_hbm.at[a, 0, pl.ds(w, CHUNK)], idx_g[b],
                         sem_i.at[b])
        pltpu.async_copy(rows_hbm.at[a, 0, pl.ds(w, CHUNK)], idx_s[b],
                         sem_i.at[b])
        pltpu.async_copy(vals_hbm.at[pl.ds(k * CHUNK, CHUNK)],
                         vals_r[b], sem_i.at[b])

    def wait_idx(b):
        pltpu.make_async_copy(cols_hbm.at[0, 0, pl.ds(0, CHUNK)], idx_g[b],
                              sem_i.at[b]).wait()
        pltpu.make_async_copy(rows_hbm.at[0, 0, pl.ds(0, CHUNK)], idx_s[b],
                              sem_i.at[b]).wait()
        pltpu.make_async_copy(vals_hbm.at[pl.ds(0, CHUNK)],
                              vals_r[b], sem_i.at[b]).wait()

    def issue_gather(bi, bb):
        pltpu.async_copy(g_hbm.at[idx_g[bi]], buf[bb], sem_g.at[bb])

    def wait_gather(bi, bb):
        pltpu.make_async_copy(g_hbm.at[idx_g[bi]], buf[bb],
                              sem_g.at[bb]).wait()

    def issue_scatter(bi, bb):
        pltpu.async_copy(buf[bb], shared.at[idx_s[bi]], sem_s.at[bb],
                         add=True)

    def wait_scatter(bi, bb):
        pltpu.make_async_copy(buf[bb], shared.at[idx_s[bi]],
                              sem_s.at[bb]).wait()

    def scale(bi, bb):
        vr = vals_r[bi]
        bf = buf[bb]

        @plsc.parallel_loop(0, CHUNK // LANES, unroll=1)
        def _(g):
            v16 = vr[pl.ds(g * LANES, LANES)]
            for j in range(LANES):
                e = g * LANES + j
                v = v16.at[jnp.full((LANES,), j, jnp.int32)].get(
                    mode="promise_in_bounds")
                for l in range(D // LANES):
                    bf[e, pl.ds(l * LANES, LANES)] = (
                        bf[e, pl.ds(l * LANES, LANES)] * v)

    issue_idx(0, 0)
    issue_idx(1, 1)
    wait_idx(0)
    issue_gather(0, 0)

    def group(g, carry):
        q0 = g * NBUF
        for j in range(NBUF):
            q = q0 + j
            b = j
            bm2 = (j - 2) % NBUF
            bp1 = (j + 1) % NBUF
            bp2 = (j + 2) % NBUF

            @pl.when(q >= 2)
            def _():
                wait_scatter(bm2, bm2)

            @pl.when(q + 2 < CPT)
            def _():
                issue_idx(q + 2, bp2)

            @pl.when(q + 1 < CPT)
            def _():
                wait_idx(bp1)
                issue_gather(bp1, bp1)

            wait_gather(b, b)
            scale(b, b)
            issue_scatter(b, b)
        return carry

    lax.fori_loop(0, CPT // NBUF, group, 0)
    wait_scatter((CPT - 2) % NBUF, (CPT - 2) % NBUF)
    wait_scatter((CPT - 1) % NBUF, (CPT - 1) % NBUF)

    @pl.when(t < NREM)
    def _():
        issue_idx(CPT, 0)
        wait_idx(0)
        issue_gather(0, 0)
        wait_gather(0, 0)
        scale(0, 0)
        issue_scatter(0, 0)
        wait_scatter(0, 0)

    plsc.subcore_barrier()
    pltpu.sync_copy(shared.at[pl.ds(s * RPT, RPT)],
                    out_hbm.at[c, pl.ds(s * RPT, RPT)])

    @pl.when(s == 0)
    def _():
        pltpu.sync_copy(shared.at[pl.ds(NS * RPT, REM)],
                        out_hbm.at[c, pl.ds(NS * RPT, REM)])


_sc_spmm = pl.kernel(
    _sc_body,
    out_type=jax.ShapeDtypeStruct((NC, N, D), jnp.float32),
    mesh=plsc.VectorSubcoreMesh(core_axis_name="c", subcore_axis_name="s",
                                num_cores=NC, num_subcores=NS),
    scratch_types=(
        [pltpu.VMEM((CHUNK,), jnp.int32) for _ in range(NBUF)]
        + [pltpu.VMEM((CHUNK,), jnp.int32) for _ in range(NBUF)]
        + [pltpu.VMEM((CHUNK,), jnp.float32) for _ in range(NBUF)]
        + [pltpu.VMEM((CHUNK, D), jnp.float32) for _ in range(NBUF)]
        + [pltpu.MemorySpace.VMEM_SHARED((N, D), jnp.float32)]
        + [pltpu.SemaphoreType.DMA((NBUF,)) for _ in range(3)]
    ),
)


def kernel(A_indices, A_values, H, weight, att_weight):
    att = att_weight.mean(axis=1)
    score = jax.nn.softmax(att, axis=1)
    M = jnp.einsum("ca,cij->aij", score, weight) / NUM_C

    G = _tc_matmul(H, M)
    rows, cols = _tc_repack(A_indices)

    zeros = jnp.zeros((N, D), jnp.float32)
    partials = _sc_spmm(G, rows, cols, A_values.reshape(-1), zeros)
    return _tc_add(partials)


# speedup vs baseline: 15.3355x; 1.0000x over previous
import jax
import jax.numpy as jnp
from jax import lax
from jax.experimental import pallas as pl
from jax.experimental.pallas import tpu as pltpu
from jax.experimental.pallas import tpu_sc as plsc

N = 10000
E = 320000
D = 128
NUM_A = 3
NUM_C = 2

NC = 2
NS = 16
LANES = 16
NW = NC * NS

CHUNK = 128
EA = NUM_A * E
NCH = EA // CHUNK
CPA = E // CHUNK
NBUF = 3
CPT = NCH // NW
NREM = NCH - CPT * NW
RPT = 624
REM = N - NS * RPT


def _mm_body(h_ref, m_ref, o_ref):
    o_ref[...] = jnp.dot(h_ref[...], m_ref[0],
                         preferred_element_type=jnp.float32)


def _tc_matmul(H, M, bn=2000):
    nb = N // bn
    return pl.pallas_call(
        _mm_body,
        grid=(nb, NUM_A),
        in_specs=[
            pl.BlockSpec((bn, D), lambda j, a: (j, 0)),
            pl.BlockSpec((1, D, D), lambda j, a: (a, 0, 0)),
        ],
        out_specs=pl.BlockSpec((bn, D), lambda j, a: (a * nb + j, 0)),
        out_shape=jax.ShapeDtypeStruct((NUM_A * N, D), jnp.float32),
    )(H, M)


def _repack_body(ai_ref, rows_ref, cols_ref):
    a = pl.program_id(0)
    rows_ref[0, 0] = ai_ref[0, 0]
    cols_ref[0, 0] = ai_ref[0, 1] + a * N


def _tc_repack(A_indices, ne=10):
    eb = E // ne
    i32 = jnp.int32
    return pl.pallas_call(
        _repack_body,
        grid=(NUM_A, ne),
        in_specs=[pl.BlockSpec((1, 2, eb), lambda a, j: (a, 0, j))],
        out_specs=[pl.BlockSpec((1, 1, eb), lambda a, j: (a, 0, j))] * 2,
        out_shape=[jax.ShapeDtypeStruct((NUM_A, 1, E), i32),
                   jax.ShapeDtypeStruct((NUM_A, 1, E), i32)],
    )(A_indices)


def _add_body(a_ref, b_ref, o_ref):
    o_ref[...] = a_ref[0] + b_ref[0]


def _tc_add(partials, bn=2000):
    return pl.pallas_call(
        _add_body,
        grid=(N // bn,),
        in_specs=[
            pl.BlockSpec((1, bn, D), lambda j: (0, j, 0)),
            pl.BlockSpec((1, bn, D), lambda j: (1, j, 0)),
        ],
        out_specs=pl.BlockSpec((bn, D), lambda j: (j, 0)),
        out_shape=jax.ShapeDtypeStruct((N, D), jnp.float32),
    )(partials, partials)


def _sc_body(g_hbm, rows_hbm, cols_hbm, vals_hbm, zeros_hbm, out_hbm,
             ig0, ig1, ig2, is0, is1, is2, vr0, vr1, vr2, bf0, bf1, bf2,
             dummy3, shared, sem_i, sem_g, sem_s):
    idx_g = [ig0, ig1, ig2]
    idx_s = [is0, is1, is2]
    vals_r = [vr0, vr1, vr2]
    buf = [bf0, bf1, bf2]
    c = lax.axis_index("c")
    s = lax.axis_index("s")
    t = c * NS + s

    pltpu.sync_copy(zeros_hbm.at[pl.ds(s * RPT, RPT)],
                    shared.at[pl.ds(s * RPT, RPT)])

    @pl.when(s == 0)
    def _():
        pltpu.sync_copy(zeros_hbm.at[pl.ds(NS * RPT, REM)],
                        shared.at[pl.ds(NS * RPT, REM)])

    plsc.subcore_barrier()

    def issue_idx(q, b):
        k = q * NW + t
        a = ((k >= CPA).astype(jnp.int32)
             + (k >= 2 * CPA).astype(jnp.int32))
        w = (k - a * CPA) * CHUNK
        pltpu.async_copy(cols_hbm.at[a, 0, pl.ds(w, CHUNK)], idx_g[b],
                         sem_i.at[b])
        pltpu.async_copy(rows_hbm.at[a, 0, pl.ds(w, CHUNK)], idx_s[b],
                         sem_i.at[b])
        pltpu.async_copy(vals_hbm.at[pl.ds(k * CHUNK, CHUNK)],
                         vals_r[b], sem_i.at[b])

    def wait_idx(b):
        pltpu.make_async_copy(vals_hbm.at[pl.ds(0, 3 * CHUNK)], dummy3,
                              sem_i.at[b]).wait()

    def issue_gather(bi, bb):
        pltpu.async_copy(g_hbm.at[idx_g[bi]], buf[bb], sem_g.at[bb])

    def wait_gather(bi, bb):
        pltpu.make_async_copy(g_hbm.at[idx_g[bi]], buf[bb],
                              sem_g.at[bb]).wait()

    def issue_scatter(bi, bb):
        pltpu.async_copy(buf[bb], shared.at[idx_s[bi]], sem_s.at[bb],
                         add=True)

    def wait_scatter(bi, bb):
        pltpu.make_async_copy(buf[bb], shared.at[idx_s[bi]],
                              sem_s.at[bb]).wait()

    def scale(bi, bb):
        vr = vals_r[bi]
        bf = buf[bb]

        @plsc.parallel_loop(0, CHUNK // LANES, unroll=1)
        def _(g):
            v16 = vr[pl.ds(g * LANES, LANES)]
            for j in range(LANES):
                e = g * LANES + j
                v = v16.at[jnp.full((LANES,), j, jnp.int32)].get(
                    mode="promise_in_bounds")
                for l in range(D // LANES):
                    bf[e, pl.ds(l * LANES, LANES)] = (
                        bf[e, pl.ds(l * LANES, LANES)] * v)

    issue_idx(0, 0)
    issue_idx(1, 1)
    wait_idx(0)
    issue_gather(0, 0)

    def group(g, carry):
        q0 = g * NBUF
        for j in range(NBUF):
            q = q0 + j
            b = j
            bm2 = (j - 2) % NBUF
            bp1 = (j + 1) % NBUF
            bp2 = (j + 2) % NBUF

            @pl.when(q >= 2)
            def _():
                wait_scatter(bm2, bm2)

            @pl.when(q + 2 < CPT)
            def _():
                issue_idx(q + 2, bp2)

            @pl.when(q + 1 < CPT)
            def _():
                wait_idx(bp1)
                issue_gather(bp1, bp1)

            wait_gather(b, b)
            scale(b, b)
            issue_scatter(b, b)
        return carry

    lax.fori_loop(0, CPT // NBUF, group, 0)
    wait_scatter((CPT - 2) % NBUF, (CPT - 2) % NBUF)
    wait_scatter((CPT - 1) % NBUF, (CPT - 1) % NBUF)

    @pl.when(t < NREM)
    def _():
        issue_idx(CPT, 0)
        wait_idx(0)
        issue_gather(0, 0)
        wait_gather(0, 0)
        scale(0, 0)
        issue_scatter(0, 0)
        wait_scatter(0, 0)

    plsc.subcore_barrier()
    pltpu.sync_copy(shared.at[pl.ds(s * RPT, RPT)],
                    out_hbm.at[c, pl.ds(s * RPT, RPT)])

    @pl.when(s == 0)
    def _():
        pltpu.sync_copy(shared.at[pl.ds(NS * RPT, REM)],
                        out_hbm.at[c, pl.ds(NS * RPT, REM)])


_sc_spmm = pl.kernel(
    _sc_body,
    out_type=jax.ShapeDtypeStruct((NC, N, D), jnp.float32),
    mesh=plsc.VectorSubcoreMesh(core_axis_name="c", subcore_axis_name="s",
                                num_cores=NC, num_subcores=NS),
    scratch_types=(
        [pltpu.VMEM((CHUNK,), jnp.int32) for _ in range(NBUF)]
        + [pltpu.VMEM((CHUNK,), jnp.int32) for _ in range(NBUF)]
        + [pltpu.VMEM((CHUNK,), jnp.float32) for _ in range(NBUF)]
        + [pltpu.VMEM((CHUNK, D), jnp.float32) for _ in range(NBUF)]
        + [pltpu.VMEM((3 * CHUNK,), jnp.float32)]
        + [pltpu.MemorySpace.VMEM_SHARED((N, D), jnp.float32)]
        + [pltpu.SemaphoreType.DMA((NBUF,)) for _ in range(3)]
    ),
)


def kernel(A_indices, A_values, H, weight, att_weight):
    att = att_weight.mean(axis=1)
    score = jax.nn.softmax(att, axis=1)
    M = jnp.einsum("ca,cij->aij", score, weight) / NUM_C

    G = _tc_matmul(H, M)
    rows, cols = _tc_repack(A_indices)

    zeros = jnp.zeros((N, D), jnp.float32)
    partials = _sc_spmm(G, rows, cols, A_values.reshape(-1), zeros)
    return _tc_add(partials)


# speedup vs baseline: 15.8792x; 1.0355x over previous
import jax
import jax.numpy as jnp
from jax import lax
from jax.experimental import pallas as pl
from jax.experimental.pallas import tpu as pltpu
from jax.experimental.pallas import tpu_sc as plsc

N = 10000
E = 320000
D = 128
NUM_A = 3
NUM_C = 2

NC = 2
NS = 16
LANES = 16
NW = NC * NS

CHUNK = 128
EA = NUM_A * E
NCH = EA // CHUNK
CPA = E // CHUNK
NBUF = 3
CPT = NCH // NW
NREM = NCH - CPT * NW
RPT = 624
REM = N - NS * RPT


def _mm_body(h_ref, m_ref, ai_ref, o_ref, rows_ref, cols_ref):
    a = pl.program_id(1)
    o_ref[...] = jnp.dot(h_ref[...], m_ref[0],
                         preferred_element_type=jnp.float32)
    rows_ref[0, 0] = ai_ref[0, 0]
    cols_ref[0, 0] = ai_ref[0, 1] + a * N


def _tc_matmul_repack(H, M, A_indices, bn=2000):
    nb = N // bn
    eb = E // nb
    i32 = jnp.int32
    return pl.pallas_call(
        _mm_body,
        grid=(nb, NUM_A),
        in_specs=[
            pl.BlockSpec((bn, D), lambda j, a: (j, 0)),
            pl.BlockSpec((1, D, D), lambda j, a: (a, 0, 0)),
            pl.BlockSpec((1, 2, eb), lambda j, a: (a, 0, j)),
        ],
        out_specs=[
            pl.BlockSpec((bn, D), lambda j, a: (a * nb + j, 0)),
            pl.BlockSpec((1, 1, eb), lambda j, a: (a, 0, j)),
            pl.BlockSpec((1, 1, eb), lambda j, a: (a, 0, j)),
        ],
        out_shape=[jax.ShapeDtypeStruct((NUM_A * N, D), jnp.float32),
                   jax.ShapeDtypeStruct((NUM_A, 1, E), i32),
                   jax.ShapeDtypeStruct((NUM_A, 1, E), i32)],
    )(H, M, A_indices)


def _add_body(a_ref, b_ref, o_ref):
    o_ref[...] = a_ref[0] + b_ref[0]


def _tc_add(partials, bn=2000):
    return pl.pallas_call(
        _add_body,
        grid=(N // bn,),
        in_specs=[
            pl.BlockSpec((1, bn, D), lambda j: (0, j, 0)),
            pl.BlockSpec((1, bn, D), lambda j: (1, j, 0)),
        ],
        out_specs=pl.BlockSpec((bn, D), lambda j: (j, 0)),
        out_shape=jax.ShapeDtypeStruct((N, D), jnp.float32),
    )(partials, partials)


def _sc_body(g_hbm, rows_hbm, cols_hbm, vals_hbm, zeros_hbm, out_hbm,
             ig0, ig1, ig2, is0, is1, is2, vr0, vr1, vr2, bf0, bf1, bf2,
             dummy3, shared, sem_i, sem_g, sem_s):
    idx_g = [ig0, ig1, ig2]
    idx_s = [is0, is1, is2]
    vals_r = [vr0, vr1, vr2]
    buf = [bf0, bf1, bf2]
    c = lax.axis_index("c")
    s = lax.axis_index("s")
    t = c * NS + s

    pltpu.sync_copy(zeros_hbm.at[pl.ds(s * RPT, RPT)],
                    shared.at[pl.ds(s * RPT, RPT)])

    @pl.when(s == 0)
    def _():
        pltpu.sync_copy(zeros_hbm.at[pl.ds(NS * RPT, REM)],
                        shared.at[pl.ds(NS * RPT, REM)])

    plsc.subcore_barrier()

    def issue_idx(q, b):
        k = q * NW + t
        a = ((k >= CPA).astype(jnp.int32)
             + (k >= 2 * CPA).astype(jnp.int32))
        w = (k - a * CPA) * CHUNK
        pltpu.async_copy(cols_hbm.at[a, 0, pl.ds(w, CHUNK)], idx_g[b],
                         sem_i.at[b])
        pltpu.async_copy(rows_hbm.at[a, 0, pl.ds(w, CHUNK)], idx_s[b],
                         sem_i.at[b])
        pltpu.async_copy(vals_hbm.at[pl.ds(k * CHUNK, CHUNK)],
                         vals_r[b], sem_i.at[b])

    def wait_idx(b):
        pltpu.make_async_copy(vals_hbm.at[pl.ds(0, 3 * CHUNK)], dummy3,
                              sem_i.at[b]).wait()

    def issue_gather(bi, bb):
        pltpu.async_copy(g_hbm.at[idx_g[bi]], buf[bb], sem_g.at[bb])

    def wait_gather(bi, bb):
        pltpu.make_async_copy(g_hbm.at[idx_g[bi]], buf[bb],
                              sem_g.at[bb]).wait()

    def issue_scatter(bi, bb):
        pltpu.async_copy(buf[bb], shared.at[idx_s[bi]], sem_s.at[bb],
                         add=True)

    def wait_scatter(bi, bb):
        pltpu.make_async_copy(buf[bb], shared.at[idx_s[bi]],
                              sem_s.at[bb]).wait()

    def scale(bi, bb):
        vr = vals_r[bi]
        bf = buf[bb]

        @plsc.parallel_loop(0, CHUNK // LANES, unroll=1)
        def _(g):
            v16 = vr[pl.ds(g * LANES, LANES)]
            for j in range(LANES):
                e = g * LANES + j
                v = v16.at[jnp.full((LANES,), j, jnp.int32)].get(
                    mode="promise_in_bounds")
                for l in range(D // LANES):
                    bf[e, pl.ds(l * LANES, LANES)] = (
                        bf[e, pl.ds(l * LANES, LANES)] * v)

    issue_idx(0, 0)
    issue_idx(1, 1)
    wait_idx(0)
    issue_gather(0, 0)

    def group(g, carry):
        q0 = g * NBUF
        for j in range(NBUF):
            q = q0 + j
            b = j
            bm2 = (j - 2) % NBUF
            bp1 = (j + 1) % NBUF
            bp2 = (j + 2) % NBUF

            @pl.when(q >= 2)
            def _():
                wait_scatter(bm2, bm2)

            @pl.when(q + 2 < CPT)
            def _():
                issue_idx(q + 2, bp2)

            @pl.when(q + 1 < CPT)
            def _():
                wait_idx(bp1)
                issue_gather(bp1, bp1)

            wait_gather(b, b)
            scale(b, b)
            issue_scatter(b, b)
        return carry

    lax.fori_loop(0, CPT // NBUF, group, 0)
    wait_scatter((CPT - 2) % NBUF, (CPT - 2) % NBUF)
    wait_scatter((CPT - 1) % NBUF, (CPT - 1) % NBUF)

    @pl.when(t < NREM)
    def _():
        issue_idx(CPT, 0)
        wait_idx(0)
        issue_gather(0, 0)
        wait_gather(0, 0)
        scale(0, 0)
        issue_scatter(0, 0)
        wait_scatter(0, 0)

    plsc.subcore_barrier()
    pltpu.sync_copy(shared.at[pl.ds(s * RPT, RPT)],
                    out_hbm.at[c, pl.ds(s * RPT, RPT)])

    @pl.when(s == 0)
    def _():
        pltpu.sync_copy(shared.at[pl.ds(NS * RPT, REM)],
                        out_hbm.at[c, pl.ds(NS * RPT, REM)])


_sc_spmm = pl.kernel(
    _sc_body,
    out_type=jax.ShapeDtypeStruct((NC, N, D), jnp.float32),
    mesh=plsc.VectorSubcoreMesh(core_axis_name="c", subcore_axis_name="s",
                                num_cores=NC, num_subcores=NS),
    scratch_types=(
        [pltpu.VMEM((CHUNK,), jnp.int32) for _ in range(NBUF)]
        + [pltpu.VMEM((CHUNK,), jnp.int32) for _ in range(NBUF)]
        + [pltpu.VMEM((CHUNK,), jnp.float32) for _ in range(NBUF)]
        + [pltpu.VMEM((CHUNK, D), jnp.float32) for _ in range(NBUF)]
        + [pltpu.VMEM((3 * CHUNK,), jnp.float32)]
        + [pltpu.MemorySpace.VMEM_SHARED((N, D), jnp.float32)]
        + [pltpu.SemaphoreType.DMA((NBUF,)) for _ in range(3)]
    ),
)


def kernel(A_indices, A_values, H, weight, att_weight):
    att = att_weight.mean(axis=1)
    score = jax.nn.softmax(att, axis=1)
    M = jnp.einsum("ca,cij->aij", score, weight) / NUM_C

    G, rows, cols = _tc_matmul_repack(H, M, A_indices)

    zeros = jnp.zeros((N, D), jnp.float32)
    partials = _sc_spmm(G, rows, cols, A_values.reshape(-1), zeros)
    return _tc_add(partials)


# speedup vs baseline: 16.1843x; 1.0192x over previous
import jax
import jax.numpy as jnp
from jax import lax
from jax.experimental import pallas as pl
from jax.experimental.pallas import tpu as pltpu
from jax.experimental.pallas import tpu_sc as plsc

N = 10000
E = 320000
D = 128
NUM_A = 3
NUM_C = 2

NC = 2
NS = 16
LANES = 16
NW = NC * NS

CHUNK = 128
EA = NUM_A * E
NCH = EA // CHUNK
CPA = E // CHUNK
NBUF = 3
CPT = NCH // NW
NREM = NCH - CPT * NW
RPT = 624
REM = N - NS * RPT


def _mm_body(h_ref, m_ref, ai_ref, av_ref, o_ref, rows_ref, cols_ref,
             vals_ref, z_ref):
    a = pl.program_id(1)
    o_ref[...] = jnp.dot(h_ref[...], m_ref[0],
                         preferred_element_type=jnp.float32)
    rows_ref[0, 0] = ai_ref[0, 0]
    cols_ref[0, 0] = ai_ref[0, 1] + a * N
    vals_ref[0, 0] = av_ref[a]
    z_ref[...] = jnp.zeros_like(z_ref)


def _tc_matmul_repack(H, M, A_indices, A_values, bn=2000):
    nb = N // bn
    eb = E // nb
    i32 = jnp.int32
    return pl.pallas_call(
        _mm_body,
        grid=(nb, NUM_A),
        in_specs=[
            pl.BlockSpec((bn, D), lambda j, a: (j, 0)),
            pl.BlockSpec((1, D, D), lambda j, a: (a, 0, 0)),
            pl.BlockSpec((1, 2, eb), lambda j, a: (a, 0, j)),
            pl.BlockSpec((NUM_A, eb), lambda j, a: (0, j)),
        ],
        out_specs=[
            pl.BlockSpec((bn, D), lambda j, a: (a * nb + j, 0)),
            pl.BlockSpec((1, 1, eb), lambda j, a: (a, 0, j)),
            pl.BlockSpec((1, 1, eb), lambda j, a: (a, 0, j)),
            pl.BlockSpec((1, 1, eb), lambda j, a: (a, 0, j)),
            pl.BlockSpec((bn, D), lambda j, a: (j, 0)),
        ],
        out_shape=[jax.ShapeDtypeStruct((NUM_A * N, D), jnp.float32),
                   jax.ShapeDtypeStruct((NUM_A, 1, E), i32),
                   jax.ShapeDtypeStruct((NUM_A, 1, E), i32),
                   jax.ShapeDtypeStruct((NUM_A, 1, E), jnp.float32),
                   jax.ShapeDtypeStruct((N, D), jnp.float32)],
    )(H, M, A_indices, A_values)


def _add_body(a_ref, b_ref, o_ref):
    o_ref[...] = a_ref[0] + b_ref[0]


def _tc_add(partials, bn=2000):
    return pl.pallas_call(
        _add_body,
        grid=(N // bn,),
        in_specs=[
            pl.BlockSpec((1, bn, D), lambda j: (0, j, 0)),
            pl.BlockSpec((1, bn, D), lambda j: (1, j, 0)),
        ],
        out_specs=pl.BlockSpec((bn, D), lambda j: (j, 0)),
        out_shape=jax.ShapeDtypeStruct((N, D), jnp.float32),
    )(partials, partials)


def _sc_body(g_hbm, rows_hbm, cols_hbm, vals_hbm, zeros_hbm, out_hbm,
             ig0, ig1, ig2, is0, is1, is2, vr0, vr1, vr2, bf0, bf1, bf2,
             dummy3, shared, sem_i, sem_g, sem_s):
    idx_g = [ig0, ig1, ig2]
    idx_s = [is0, is1, is2]
    vals_r = [vr0, vr1, vr2]
    buf = [bf0, bf1, bf2]
    c = lax.axis_index("c")
    s = lax.axis_index("s")
    t = c * NS + s

    pltpu.sync_copy(zeros_hbm.at[pl.ds(s * RPT, RPT)],
                    shared.at[pl.ds(s * RPT, RPT)])

    @pl.when(s == 0)
    def _():
        pltpu.sync_copy(zeros_hbm.at[pl.ds(NS * RPT, REM)],
                        shared.at[pl.ds(NS * RPT, REM)])

    plsc.subcore_barrier()

    def issue_idx(q, b):
        k = q * NW + t
        a = ((k >= CPA).astype(jnp.int32)
             + (k >= 2 * CPA).astype(jnp.int32))
        w = (k - a * CPA) * CHUNK
        pltpu.async_copy(cols_hbm.at[a, 0, pl.ds(w, CHUNK)], idx_g[b],
                         sem_i.at[b])
        pltpu.async_copy(rows_hbm.at[a, 0, pl.ds(w, CHUNK)], idx_s[b],
                         sem_i.at[b])
        pltpu.async_copy(vals_hbm.at[a, 0, pl.ds(w, CHUNK)],
                         vals_r[b], sem_i.at[b])

    def wait_idx(b):
        pltpu.make_async_copy(vals_hbm.at[0, 0, pl.ds(0, 3 * CHUNK)], dummy3,
                              sem_i.at[b]).wait()

    def issue_gather(bi, bb):
        pltpu.async_copy(g_hbm.at[idx_g[bi]], buf[bb], sem_g.at[bb])

    def wait_gather(bi, bb):
        pltpu.make_async_copy(g_hbm.at[idx_g[bi]], buf[bb],
                              sem_g.at[bb]).wait()

    def issue_scatter(bi, bb):
        pltpu.async_copy(buf[bb], shared.at[idx_s[bi]], sem_s.at[bb],
                         add=True)

    def wait_scatter(bi, bb):
        pltpu.make_async_copy(buf[bb], shared.at[idx_s[bi]],
                              sem_s.at[bb]).wait()

    def scale(bi, bb):
        vr = vals_r[bi]
        bf = buf[bb]

        @plsc.parallel_loop(0, CHUNK // LANES, unroll=1)
        def _(g):
            v16 = vr[pl.ds(g * LANES, LANES)]
            for j in range(LANES):
                e = g * LANES + j
                v = v16.at[jnp.full((LANES,), j, jnp.int32)].get(
                    mode="promise_in_bounds")
                for l in range(D // LANES):
                    bf[e, pl.ds(l * LANES, LANES)] = (
                        bf[e, pl.ds(l * LANES, LANES)] * v)

    issue_idx(0, 0)
    issue_idx(1, 1)
    wait_idx(0)
    issue_gather(0, 0)

    def group(g, carry):
        q0 = g * NBUF
        for j in range(NBUF):
            q = q0 + j
            b = j
            bm2 = (j - 2) % NBUF
            bp1 = (j + 1) % NBUF
            bp2 = (j + 2) % NBUF

            @pl.when(q >= 2)
            def _():
                wait_scatter(bm2, bm2)

            @pl.when(q + 2 < CPT)
            def _():
                issue_idx(q + 2, bp2)

            @pl.when(q + 1 < CPT)
            def _():
                wait_idx(bp1)
                issue_gather(bp1, bp1)

            wait_gather(b, b)
            scale(b, b)
            issue_scatter(b, b)
        return carry

    lax.fori_loop(0, CPT // NBUF, group, 0)
    wait_scatter((CPT - 2) % NBUF, (CPT - 2) % NBUF)
    wait_scatter((CPT - 1) % NBUF, (CPT - 1) % NBUF)

    @pl.when(t < NREM)
    def _():
        issue_idx(CPT, 0)
        wait_idx(0)
        issue_gather(0, 0)
        wait_gather(0, 0)
        scale(0, 0)
        issue_scatter(0, 0)
        wait_scatter(0, 0)

    plsc.subcore_barrier()
    pltpu.sync_copy(shared.at[pl.ds(s * RPT, RPT)],
                    out_hbm.at[c, pl.ds(s * RPT, RPT)])

    @pl.when(s == 0)
    def _():
        pltpu.sync_copy(shared.at[pl.ds(NS * RPT, REM)],
                        out_hbm.at[c, pl.ds(NS * RPT, REM)])


_sc_spmm = pl.kernel(
    _sc_body,
    out_type=jax.ShapeDtypeStruct((NC, N, D), jnp.float32),
    mesh=plsc.VectorSubcoreMesh(core_axis_name="c", subcore_axis_name="s",
                                num_cores=NC, num_subcores=NS),
    scratch_types=(
        [pltpu.VMEM((CHUNK,), jnp.int32) for _ in range(NBUF)]
        + [pltpu.VMEM((CHUNK,), jnp.int32) for _ in range(NBUF)]
        + [pltpu.VMEM((CHUNK,), jnp.float32) for _ in range(NBUF)]
        + [pltpu.VMEM((CHUNK, D), jnp.float32) for _ in range(NBUF)]
        + [pltpu.VMEM((3 * CHUNK,), jnp.float32)]
        + [pltpu.MemorySpace.VMEM_SHARED((N, D), jnp.float32)]
        + [pltpu.SemaphoreType.DMA((NBUF,)) for _ in range(3)]
    ),
)


def kernel(A_indices, A_values, H, weight, att_weight):
    att = att_weight.mean(axis=1)
    score = jax.nn.softmax(att, axis=1)
    M = jnp.einsum("ca,cij->aij", score, weight) / NUM_C

    G, rows, cols, vals, zeros = _tc_matmul_repack(H, M, A_indices,
                                                   A_values)
    partials = _sc_spmm(G, rows, cols, vals, zeros)
    return _tc_add(partials)
